# Initial kernel scaffold; baseline (speedup 1.0000x reference)
import jax
import jax.numpy as jnp
from jax.experimental import pallas as pl


def kernel(outputs, target_classes, clusters):
    raise NotImplementedError("write your pallas kernel here")


# speedup vs baseline: 1.5535x; 1.5535x over previous
import functools

import jax
import jax.numpy as jnp
from jax.experimental import pallas as pl
from jax.experimental.pallas import tpu as pltpu

NUM_CLASSES = 10
NUM_PROT = 512
NUM_FEAT = 64
BATCH = 4096
ALPHA = 5.0
BETA = 5.0

TILE_B = 512
GRID = BATCH // TILE_B


def _cce_kernel(x_ref, tc_ref, ct_ref, out_ref, acc_ref):
    i = pl.program_id(0)

    @pl.when(i == 0)
    def _():
        acc_ref[0, 0] = 0.0

    x = x_ref[...]
    ct = ct_ref[...]
    xc = jnp.dot(x, ct, preferred_element_type=jnp.float32)
    x2 = jnp.sum(x * x, axis=1)
    c2 = jnp.sum(ct * ct, axis=0)
    sq = x2[:, None] + c2[None, :] - 2.0 * xc
    sq = jnp.maximum(sq, 0.0)

    tc = tc_ref[0, 0, :]
    sel = jnp.full((TILE_B,), jnp.inf, dtype=jnp.float32)
    for c in range(NUM_CLASSES):
        m_c = jnp.min(sq[:, c * NUM_PROT:(c + 1) * NUM_PROT], axis=1)
        sel = jnp.where(tc == c, m_c, sel)
    acc_ref[0, 0] += jnp.sum(sel)

    @pl.when(i == GRID - 1)
    def _():
        t = acc_ref[0, 0] / (BATCH * NUM_FEAT)
        out_ref[0, 0] = ALPHA * t + BETA * (1.0 - t)


@jax.jit
def kernel(outputs, target_classes, clusters):
    ct = jnp.transpose(clusters.reshape(NUM_CLASSES * NUM_PROT, NUM_FEAT))
    tc = target_classes.astype(jnp.int32).reshape(GRID, 1, TILE_B)
    out = pl.pallas_call(
        _cce_kernel,
        grid=(GRID,),
        in_specs=[
            pl.BlockSpec((TILE_B, NUM_FEAT), lambda i: (i, 0)),
            pl.BlockSpec((1, 1, TILE_B), lambda i: (i, 0, 0)),
            pl.BlockSpec((NUM_FEAT, NUM_CLASSES * NUM_PROT), lambda i: (0, 0)),
        ],
        out_specs=pl.BlockSpec(memory_space=pltpu.SMEM),
        out_shape=jax.ShapeDtypeStruct((1, 1), jnp.float32),
        scratch_shapes=[pltpu.SMEM((1, 1), jnp.float32)],
    )(outputs, tc, ct)
    return out[0, 0]
